# baseline (speedup 1.0000x reference)
import functools

import jax
import jax.numpy as jnp
from jax.experimental import pallas as pl

_EPS = 1e-08
_UPDATE_SCALE = 1.0
_DIST_THRESHOLD = 3.0
_W_INJECT = 1.0
_W_PIXEL = 1.0
_B = 4
_H = 64
_W = 64
_BIG = 1e20
_INVALID = 1e12


def _zc_grids(sdf):
    v1 = sdf[..., :-1, :]
    v2 = sdf[..., 1:, :]
    ii = jax.lax.broadcasted_iota(jnp.int32, v1.shape, v1.ndim - 2).astype(jnp.float32)
    c1 = v1 == 0.0
    c2 = jnp.logical_and(~c1, v2 == 0.0)
    c3 = jnp.logical_and(jnp.logical_and(~c1, ~c2), v1 * v2 < 0.0)
    alpha = jnp.abs(v1) / (jnp.abs(v1) + jnp.abs(v2) + _EPS)
    r_v = jnp.where(c1, ii, jnp.where(c2, ii + 1.0, ii + alpha))
    m_v = c1 | c2 | c3
    h1 = sdf[..., :, :-1]
    h2 = sdf[..., :, 1:]
    jj = jax.lax.broadcasted_iota(jnp.int32, h1.shape, h1.ndim - 1).astype(jnp.float32)
    d1 = h1 == 0.0
    d2 = jnp.logical_and(~d1, h2 == 0.0)
    d3 = jnp.logical_and(jnp.logical_and(~d1, ~d2), h1 * h2 < 0.0)
    beta = jnp.abs(h1) / (jnp.abs(h1) + jnp.abs(h2) + _EPS)
    c_h = jnp.where(d1, jj, jnp.where(d2, jj + 1.0, jj + beta))
    m_h = d1 | d2 | d3
    return (r_v, m_v), (c_h, m_h)


def _pad2(a, top, bottom, left, right, val):
    cfg = [(0, 0, 0)] * (a.ndim - 2) + [(top, bottom, 0), (left, right, 0)]
    return jax.lax.pad(a, jnp.float32(val), cfg)


def _nn_kernel(pred_ref, gt_ref, dv_ref, grv_ref, gcv_ref,
               dh_ref, grh_ref, gch_ref, vpack_ref, hpack_ref):
    pred = pred_ref[...]
    gt = gt_ref[...]
    (gv_r, gv_m), (gh_c, gh_m) = _zc_grids(gt)
    (pv_r, _), (ph_c, _) = _zc_grids(pred)

    GV = _pad2(jnp.where(gv_m, gv_r, _INVALID), 4, 4, 3, 4, _INVALID)
    GH = _pad2(jnp.where(gh_m, gh_c, _INVALID), 3, 4, 4, 4, _INVALID)

    iotaI_v = jax.lax.broadcasted_iota(jnp.int32, (_B, _H - 1, _W), 1).astype(jnp.float32)
    iotaJ_v = jax.lax.broadcasted_iota(jnp.int32, (_B, _H - 1, _W), 2).astype(jnp.float32)
    iotaI_h = jax.lax.broadcasted_iota(jnp.int32, (_B, _H, _W - 1), 1).astype(jnp.float32)
    iotaJ_h = jax.lax.broadcasted_iota(jnp.int32, (_B, _H, _W - 1), 2).astype(jnp.float32)

    def search(pr_r, pr_c, hh, ww, iI, iJ, vv, vh, v_aligned):
        best_d = jnp.full((_B, hh, ww), _BIG, jnp.float32)
        best_gr = jnp.zeros((_B, hh, ww), jnp.float32)
        best_gc = jnp.zeros((_B, hh, ww), jnp.float32)

        def upd(d, g_r, g_c, bd, bgr, bgc):
            better = d < bd
            return (jnp.where(better, d, bd),
                    jnp.where(better, g_r, bgr),
                    jnp.where(better, g_c, bgc))

        dlo, dhi, jlo, jhi = vv
        g_cs = {dj: iJ + jnp.float32(dj) for dj in range(jlo, jhi + 1)}
        for di in range(dlo, dhi + 1):
            for dj in range(jlo, jhi + 1):
                g_r = GV[:, 4 + di:4 + di + hh, 3 + dj:3 + dj + ww]
                g_c = g_cs[dj]
                dr = pr_r - g_r
                if v_aligned:
                    d = jnp.sqrt(dr * dr + jnp.float32(float(dj) * float(dj)))
                else:
                    dc = pr_c - g_c
                    d = jnp.sqrt(dr * dr + dc * dc)
                best_d, best_gr, best_gc = upd(d, g_r, g_c,
                                               best_d, best_gr, best_gc)
        dlo, dhi, jlo, jhi = vh
        g_rs = {di: iI + jnp.float32(di) for di in range(dlo, dhi + 1)}
        for di in range(dlo, dhi + 1):
            for dj in range(jlo, jhi + 1):
                g_c = GH[:, 3 + di:3 + di + hh, 4 + dj:4 + dj + ww]
                g_r = g_rs[di]
                dc = pr_c - g_c
                if not v_aligned:
                    d = jnp.sqrt(jnp.float32(float(di) * float(di)) + dc * dc)
                else:
                    dr = pr_r - g_r
                    d = jnp.sqrt(dr * dr + dc * dc)
                best_d, best_gr, best_gc = upd(d, g_r, g_c,
                                               best_d, best_gr, best_gc)
        return best_d, best_gr, best_gc

    bd, bgr, bgc = search(pv_r, iotaJ_v, _H - 1, _W, iotaI_v, iotaJ_v,
                          (-4, 4, -3, 3), (-3, 4, -4, 3), True)
    dv_ref[...] = bd
    grv_ref[...] = bgr
    gcv_ref[...] = bgc

    bd, bgr, bgc = search(iotaI_h, ph_c, _H, _W - 1, iotaI_h, iotaJ_h,
                          (-4, 3, -3, 4), (-3, 3, -4, 4), False)
    dh_ref[...] = bd
    grh_ref[...] = bgr
    gch_ref[...] = bgc

    grad_r = jnp.concatenate(
        [pred[:, 1:2, :] - pred[:, 0:1, :],
         (pred[:, 2:, :] - pred[:, :-2, :]) / 2.0,
         pred[:, -1:, :] - pred[:, -2:-1, :]], axis=1)
    grad_c = jnp.concatenate(
        [pred[:, :, 1:2] - pred[:, :, 0:1],
         (pred[:, :, 2:] - pred[:, :, :-2]) / 2.0,
         pred[:, :, -1:] - pred[:, :, -2:-1]], axis=2)

    eq_v = pv_r == iotaI_v + 1.0
    eq_h = ph_c == iotaJ_h + 1.0

    def corner_v(X):
        Xc = jnp.concatenate([X[:, :, 1:], X[:, :, -1:]], axis=2)
        R2 = jnp.concatenate([X[:, 2:, :], X[:, -1:, :]], axis=1)
        C2 = jnp.concatenate([Xc[:, 2:, :], Xc[:, -1:, :]], axis=1)
        return (jnp.where(eq_v, X[:, 1:, :], X[:, :-1, :]),
                jnp.where(eq_v, Xc[:, 1:, :], Xc[:, :-1, :]),
                jnp.where(eq_v, R2, X[:, 1:, :]),
                jnp.where(eq_v, C2, Xc[:, 1:, :]))

    def corner_h(X):
        Xr = jnp.concatenate([X[:, 1:, :], X[:, -1:, :]], axis=1)
        A2 = jnp.concatenate([X[:, :, 2:], X[:, :, -1:]], axis=2)
        B2 = jnp.concatenate([Xr[:, :, 2:], Xr[:, :, -1:]], axis=2)
        return (jnp.where(eq_h, X[:, :, 1:], X[:, :, :-1]),
                jnp.where(eq_h, A2, X[:, :, 1:]),
                jnp.where(eq_h, Xr[:, :, 1:], Xr[:, :, :-1]),
                jnp.where(eq_h, B2, Xr[:, :, 1:]))

    vparts = []
    hparts = []
    for X in (grad_r, grad_c, pred):
        vparts.append(corner_v(X))
        hparts.append(corner_h(X))
    vpack_ref[...] = jnp.stack(
        [vparts[x][k] for k in range(4) for x in range(3)], axis=1)
    hpack_ref[...] = jnp.stack(
        [hparts[x][k] for k in range(4) for x in range(3)], axis=1)


@functools.partial(jax.jit, static_argnames=())
def _nn_search(pred_sdf, gt_sdf):
    shp_v = jax.ShapeDtypeStruct((_B, _H - 1, _W), jnp.float32)
    shp_h = jax.ShapeDtypeStruct((_B, _H, _W - 1), jnp.float32)
    pk_v = jax.ShapeDtypeStruct((_B, 12, _H - 1, _W), jnp.float32)
    pk_h = jax.ShapeDtypeStruct((_B, 12, _H, _W - 1), jnp.float32)
    return pl.pallas_call(
        _nn_kernel,
        out_shape=(shp_v, shp_v, shp_v, shp_h, shp_h, shp_h, pk_v, pk_h),
    )(pred_sdf, gt_sdf)


def _extract_zero_crossings(sdf):
    h, w = sdf.shape
    (r_v, m_v), (c_h, m_h) = _zc_grids(sdf)
    ii2 = jnp.broadcast_to(jnp.arange(h, dtype=jnp.float32)[:, None], (h, w - 1))
    jj = jnp.broadcast_to(jnp.arange(w, dtype=jnp.float32)[None, :], (h - 1, w))
    pts_v = jnp.stack([r_v.reshape(-1), jj.reshape(-1)], axis=1)
    pts_h = jnp.stack([ii2.reshape(-1), c_h.reshape(-1)], axis=1)
    return (jnp.concatenate([pts_v, pts_h], axis=0),
            jnp.concatenate([m_v.reshape(-1), m_h.reshape(-1)], axis=0))


def _compute_normals(sdf):
    grad_r = jnp.zeros_like(sdf)
    grad_r = grad_r.at[1:-1].set((sdf[2:] - sdf[:-2]) / 2.0)
    grad_r = grad_r.at[0].set(sdf[1] - sdf[0])
    grad_r = grad_r.at[-1].set(sdf[-1] - sdf[-2])
    grad_c = jnp.zeros_like(sdf)
    grad_c = grad_c.at[:, 1:-1].set((sdf[:, 2:] - sdf[:, :-2]) / 2.0)
    grad_c = grad_c.at[:, 0].set(sdf[:, 1] - sdf[:, 0])
    grad_c = grad_c.at[:, -1].set(sdf[:, -1] - sdf[:, -2])
    return jnp.stack([grad_r, grad_c], axis=2)


def _sample_normals(corners, pts, h, w):
    r = pts[:, 0]
    c = pts[:, 1]
    r0u = jnp.floor(r).astype(jnp.int32)
    c0u = jnp.floor(c).astype(jnp.int32)
    r0 = jnp.clip(r0u, 0, h - 1)
    c0 = jnp.clip(c0u, 0, w - 1)
    ar = (r - r0.astype(jnp.float32))[:, None]
    ac = (c - c0.astype(jnp.float32))[:, None]
    Ia, Ib, Ic, Id = corners
    n = Ia * (1 - ar) * (1 - ac) + Ib * (1 - ar) * ac + Ic * ar * (1 - ac) + Id * ar * ac
    return n / (jnp.linalg.norm(n, axis=1, keepdims=True) + 1e-08)


def _sample_pred_at_positions(corners, positions):
    r = positions[:, 0]
    c = positions[:, 1]
    r0u = jnp.floor(r).astype(jnp.int32)
    c0u = jnp.floor(c).astype(jnp.int32)
    dr = (r - r0u.astype(jnp.float32))[:, None]
    dc = (c - c0u.astype(jnp.float32))[:, None]
    Ia, Ib, Ic, Id = (x[:, None] for x in corners)
    out = Ia * (1 - dr) * (1 - dc) + Ib * (1 - dr) * dc + Ic * dr * (1 - dc) + Id * dr * dc
    return out[:, 0]


def _unpack_corners(vpack, hpack):
    ncorners = []
    pcorners = []
    for k in range(4):
        parts = []
        for x in range(3):
            idx = k * 3 + x
            parts.append(jnp.concatenate(
                [vpack[idx].reshape(-1), hpack[idx].reshape(-1)], axis=0))
        ncorners.append(jnp.stack(parts[:2], axis=1))
        pcorners.append(parts[2])
    return ncorners, pcorners


def _per_image(pred, md, gt_sel, vpack, hpack):
    h, w = pred.shape
    pred_zc, pmask = _extract_zero_crossings(pred)
    nv = (h - 1) * w
    r0f = jnp.floor(pred_zc[:, 0]).astype(jnp.int32)
    c0f = jnp.floor(pred_zc[:, 1]).astype(jnp.int32)
    eq_v = r0f[:nv].reshape(h - 1, w) == jnp.arange(h - 1, dtype=jnp.int32)[:, None] + 1
    eq_h = c0f[nv:].reshape(h, w - 1) == jnp.arange(w - 1, dtype=jnp.int32)[None, :] + 1
    ncorners, pcorners = _unpack_corners(vpack, hpack)
    sampled = _sample_normals(ncorners, pred_zc, h, w)
    keep = pmask & (md <= _DIST_THRESHOLD)
    dirv = gt_sel - pred_zc
    dot = jnp.sum(dirv * sampled, axis=1) * _UPDATE_SCALE
    r = pred_zc[:, 0]
    c = pred_zc[:, 1]
    r0 = jnp.floor(r).astype(jnp.int32)
    c0 = jnp.floor(c).astype(jnp.int32)
    r1 = r0 + 1
    c1 = c0 + 1
    ar = r - r0.astype(jnp.float32)
    ac = c - c0.astype(jnp.float32)
    corners = [(r0, c0, (1 - ar) * (1 - ac)), (r0, c1, (1 - ar) * ac),
               (r1, c0, ar * (1 - ac)), (r1, c1, ar * ac)]
    vals = []
    for rr, cc, wgt in corners:
        inb = (rr >= 0) & (rr < h) & (cc >= 0) & (cc < w)
        vals.append(jnp.where(keep & inb, dot * wgt, 0.0))
    lt_v = ~eq_v
    lt_h = ~eq_h
    v1g = vals[0][:nv].reshape(h - 1, w)
    v3g = vals[2][:nv].reshape(h - 1, w)
    h1g = vals[0][nv:].reshape(h, w - 1)
    h2g = vals[1][nv:].reshape(h, w - 1)
    T1 = jnp.pad(jnp.where(eq_v, v1g, 0.0), ((1, 0), (0, 0)))
    T2 = jnp.pad(jnp.where(lt_v, v1g, 0.0), ((0, 1), (0, 0)))
    T3 = jnp.pad(jnp.where(eq_h, h1g, 0.0), ((0, 0), (1, 0)))
    T4 = jnp.pad(jnp.where(lt_h, h1g, 0.0), ((0, 0), (0, 1)))
    T5 = jnp.pad(jnp.where(lt_h, h2g, 0.0), ((0, 0), (1, 0)))
    T6 = jnp.pad(jnp.where(lt_v, v3g, 0.0), ((1, 0), (0, 0)))
    dSDF = T1 + T2 + T3 + T4 + T5 + T6
    inj = jnp.sum(pred * dSDF)
    vals = _sample_pred_at_positions(pcorners, pred_zc)
    pix = jnp.sum(jnp.where(pmask, vals, 0.0))
    return inj, pix


def kernel(pred_sdf, gt_sdf):
    dv, grv, gcv, dh, grh, gch, vpack, hpack = _nn_search(pred_sdf, gt_sdf)
    injs = []
    pixs = []
    for b in range(pred_sdf.shape[0]):
        md = jnp.concatenate([dv[b].reshape(-1), dh[b].reshape(-1)], axis=0)
        gt_sel = jnp.stack(
            [jnp.concatenate([grv[b].reshape(-1), grh[b].reshape(-1)], axis=0),
             jnp.concatenate([gcv[b].reshape(-1), gch[b].reshape(-1)], axis=0)],
            axis=1)
        inj, pix = _per_image(pred_sdf[b], md, gt_sel, vpack[b], hpack[b])
        injs.append(inj)
        pixs.append(pix)
    inject = jnp.stack(injs).mean()
    pixel = jnp.stack(pixs).mean()
    return _W_INJECT * inject + _W_PIXEL * pixel


# speedup vs baseline: 1.8231x; 1.8231x over previous
import functools

import jax
import jax.numpy as jnp
from jax.experimental import pallas as pl

_EPS = 1e-08
_UPDATE_SCALE = 1.0
_DIST_THRESHOLD = 3.0
_W_INJECT = 1.0
_W_PIXEL = 1.0
_B = 4
_H = 64
_W = 64
_BIG = 1e20
_INVALID = 1e12


def _zc_grids(sdf):
    v1 = sdf[..., :-1, :]
    v2 = sdf[..., 1:, :]
    ii = jax.lax.broadcasted_iota(jnp.int32, v1.shape, v1.ndim - 2).astype(jnp.float32)
    c1 = v1 == 0.0
    c2 = jnp.logical_and(~c1, v2 == 0.0)
    c3 = jnp.logical_and(jnp.logical_and(~c1, ~c2), v1 * v2 < 0.0)
    alpha = jnp.abs(v1) / (jnp.abs(v1) + jnp.abs(v2) + _EPS)
    r_v = jnp.where(c1, ii, jnp.where(c2, ii + 1.0, ii + alpha))
    m_v = c1 | c2 | c3
    h1 = sdf[..., :, :-1]
    h2 = sdf[..., :, 1:]
    jj = jax.lax.broadcasted_iota(jnp.int32, h1.shape, h1.ndim - 1).astype(jnp.float32)
    d1 = h1 == 0.0
    d2 = jnp.logical_and(~d1, h2 == 0.0)
    d3 = jnp.logical_and(jnp.logical_and(~d1, ~d2), h1 * h2 < 0.0)
    beta = jnp.abs(h1) / (jnp.abs(h1) + jnp.abs(h2) + _EPS)
    c_h = jnp.where(d1, jj, jnp.where(d2, jj + 1.0, jj + beta))
    m_h = d1 | d2 | d3
    return (r_v, m_v), (c_h, m_h)


def _pad2(a, top, bottom, left, right, val):
    cfg = [(0, 0, 0)] * (a.ndim - 2) + [(top, bottom, 0), (left, right, 0)]
    return jax.lax.pad(a, jnp.float32(val), cfg)


def _nn_kernel(pred_ref, gt_ref, dv_ref, grv_ref, gcv_ref,
               dh_ref, grh_ref, gch_ref, vpack_ref, hpack_ref):
    pred = pred_ref[...]
    gt = gt_ref[...]
    (gv_r, gv_m), (gh_c, gh_m) = _zc_grids(gt)
    (pv_r, _), (ph_c, _) = _zc_grids(pred)

    GV = _pad2(jnp.where(gv_m, gv_r, _INVALID), 4, 4, 3, 4, _INVALID)
    GH = _pad2(jnp.where(gh_m, gh_c, _INVALID), 3, 4, 4, 4, _INVALID)

    iotaI_v = jax.lax.broadcasted_iota(jnp.int32, (_B, _H - 1, _W), 1).astype(jnp.float32)
    iotaJ_v = jax.lax.broadcasted_iota(jnp.int32, (_B, _H - 1, _W), 2).astype(jnp.float32)
    iotaI_h = jax.lax.broadcasted_iota(jnp.int32, (_B, _H, _W - 1), 1).astype(jnp.float32)
    iotaJ_h = jax.lax.broadcasted_iota(jnp.int32, (_B, _H, _W - 1), 2).astype(jnp.float32)

    def search(pr_r, pr_c, hh, ww, iI, iJ, vv, vh, v_aligned):
        best_d = jnp.full((_B, hh, ww), _BIG, jnp.float32)
        best_gr = jnp.zeros((_B, hh, ww), jnp.float32)
        best_gc = jnp.zeros((_B, hh, ww), jnp.float32)

        def upd(d, g_r, g_c, bd, bgr, bgc):
            better = d < bd
            return (jnp.where(better, d, bd),
                    jnp.where(better, g_r, bgr),
                    jnp.where(better, g_c, bgc))

        dlo, dhi, jlo, jhi = vv
        g_cs = {dj: iJ + jnp.float32(dj) for dj in range(jlo, jhi + 1)}
        for di in range(dlo, dhi + 1):
            for dj in range(jlo, jhi + 1):
                g_r = GV[:, 4 + di:4 + di + hh, 3 + dj:3 + dj + ww]
                g_c = g_cs[dj]
                dr = pr_r - g_r
                if v_aligned:
                    d = jnp.sqrt(dr * dr + jnp.float32(float(dj) * float(dj)))
                else:
                    dc = pr_c - g_c
                    d = jnp.sqrt(dr * dr + dc * dc)
                best_d, best_gr, best_gc = upd(d, g_r, g_c,
                                               best_d, best_gr, best_gc)
        dlo, dhi, jlo, jhi = vh
        g_rs = {di: iI + jnp.float32(di) for di in range(dlo, dhi + 1)}
        for di in range(dlo, dhi + 1):
            for dj in range(jlo, jhi + 1):
                g_c = GH[:, 3 + di:3 + di + hh, 4 + dj:4 + dj + ww]
                g_r = g_rs[di]
                dc = pr_c - g_c
                if not v_aligned:
                    d = jnp.sqrt(jnp.float32(float(di) * float(di)) + dc * dc)
                else:
                    dr = pr_r - g_r
                    d = jnp.sqrt(dr * dr + dc * dc)
                best_d, best_gr, best_gc = upd(d, g_r, g_c,
                                               best_d, best_gr, best_gc)
        return best_d, best_gr, best_gc

    bd, bgr, bgc = search(pv_r, iotaJ_v, _H - 1, _W, iotaI_v, iotaJ_v,
                          (-4, 4, -3, 3), (-3, 4, -4, 3), True)
    dv_ref[...] = bd
    grv_ref[...] = bgr
    gcv_ref[...] = bgc

    bd, bgr, bgc = search(iotaI_h, ph_c, _H, _W - 1, iotaI_h, iotaJ_h,
                          (-4, 3, -3, 4), (-3, 3, -4, 4), False)
    dh_ref[...] = bd
    grh_ref[...] = bgr
    gch_ref[...] = bgc

    grad_r = jnp.concatenate(
        [pred[:, 1:2, :] - pred[:, 0:1, :],
         (pred[:, 2:, :] - pred[:, :-2, :]) / 2.0,
         pred[:, -1:, :] - pred[:, -2:-1, :]], axis=1)
    grad_c = jnp.concatenate(
        [pred[:, :, 1:2] - pred[:, :, 0:1],
         (pred[:, :, 2:] - pred[:, :, :-2]) / 2.0,
         pred[:, :, -1:] - pred[:, :, -2:-1]], axis=2)

    eq_v = pv_r == iotaI_v + 1.0
    eq_h = ph_c == iotaJ_h + 1.0

    def corner_v(X):
        Xc = jnp.concatenate([X[:, :, 1:], X[:, :, -1:]], axis=2)
        R2 = jnp.concatenate([X[:, 2:, :], X[:, -1:, :]], axis=1)
        C2 = jnp.concatenate([Xc[:, 2:, :], Xc[:, -1:, :]], axis=1)
        return (jnp.where(eq_v, X[:, 1:, :], X[:, :-1, :]),
                jnp.where(eq_v, Xc[:, 1:, :], Xc[:, :-1, :]),
                jnp.where(eq_v, R2, X[:, 1:, :]),
                jnp.where(eq_v, C2, Xc[:, 1:, :]))

    def corner_h(X):
        Xr = jnp.concatenate([X[:, 1:, :], X[:, -1:, :]], axis=1)
        A2 = jnp.concatenate([X[:, :, 2:], X[:, :, -1:]], axis=2)
        B2 = jnp.concatenate([Xr[:, :, 2:], Xr[:, :, -1:]], axis=2)
        return (jnp.where(eq_h, X[:, :, 1:], X[:, :, :-1]),
                jnp.where(eq_h, A2, X[:, :, 1:]),
                jnp.where(eq_h, Xr[:, :, 1:], Xr[:, :, :-1]),
                jnp.where(eq_h, B2, Xr[:, :, 1:]))

    vparts = []
    hparts = []
    for X in (grad_r, grad_c, pred):
        vparts.append(corner_v(X))
        hparts.append(corner_h(X))
    vpack_ref[...] = jnp.stack(
        [vparts[x][k] for k in range(4) for x in range(3)], axis=1)
    hpack_ref[...] = jnp.stack(
        [hparts[x][k] for k in range(4) for x in range(3)], axis=1)


@functools.partial(jax.jit, static_argnames=())
def _nn_search(pred_sdf, gt_sdf):
    shp_v = jax.ShapeDtypeStruct((_B, _H - 1, _W), jnp.float32)
    shp_h = jax.ShapeDtypeStruct((_B, _H, _W - 1), jnp.float32)
    pk_v = jax.ShapeDtypeStruct((_B, 12, _H - 1, _W), jnp.float32)
    pk_h = jax.ShapeDtypeStruct((_B, 12, _H, _W - 1), jnp.float32)
    return pl.pallas_call(
        _nn_kernel,
        out_shape=(shp_v, shp_v, shp_v, shp_h, shp_h, shp_h, pk_v, pk_h),
    )(pred_sdf, gt_sdf)


def _extract_zero_crossings(sdf):
    h, w = sdf.shape
    (r_v, m_v), (c_h, m_h) = _zc_grids(sdf)
    ii2 = jnp.broadcast_to(jnp.arange(h, dtype=jnp.float32)[:, None], (h, w - 1))
    jj = jnp.broadcast_to(jnp.arange(w, dtype=jnp.float32)[None, :], (h - 1, w))
    pts_v = jnp.stack([r_v.reshape(-1), jj.reshape(-1)], axis=1)
    pts_h = jnp.stack([ii2.reshape(-1), c_h.reshape(-1)], axis=1)
    return (jnp.concatenate([pts_v, pts_h], axis=0),
            jnp.concatenate([m_v.reshape(-1), m_h.reshape(-1)], axis=0))


def _compute_normals(sdf):
    grad_r = jnp.zeros_like(sdf)
    grad_r = grad_r.at[1:-1].set((sdf[2:] - sdf[:-2]) / 2.0)
    grad_r = grad_r.at[0].set(sdf[1] - sdf[0])
    grad_r = grad_r.at[-1].set(sdf[-1] - sdf[-2])
    grad_c = jnp.zeros_like(sdf)
    grad_c = grad_c.at[:, 1:-1].set((sdf[:, 2:] - sdf[:, :-2]) / 2.0)
    grad_c = grad_c.at[:, 0].set(sdf[:, 1] - sdf[:, 0])
    grad_c = grad_c.at[:, -1].set(sdf[:, -1] - sdf[:, -2])
    return jnp.stack([grad_r, grad_c], axis=2)


def _sample_normals(corners, pts, h, w):
    r = pts[:, 0]
    c = pts[:, 1]
    r0u = jnp.floor(r).astype(jnp.int32)
    c0u = jnp.floor(c).astype(jnp.int32)
    r0 = jnp.clip(r0u, 0, h - 1)
    c0 = jnp.clip(c0u, 0, w - 1)
    ar = (r - r0.astype(jnp.float32))[:, None]
    ac = (c - c0.astype(jnp.float32))[:, None]
    Ia, Ib, Ic, Id = corners
    n = Ia * (1 - ar) * (1 - ac) + Ib * (1 - ar) * ac + Ic * ar * (1 - ac) + Id * ar * ac
    return n / (jnp.linalg.norm(n, axis=1, keepdims=True) + 1e-08)


def _sample_pred_at_positions(corners, positions):
    r = positions[:, 0]
    c = positions[:, 1]
    r0u = jnp.floor(r).astype(jnp.int32)
    c0u = jnp.floor(c).astype(jnp.int32)
    dr = (r - r0u.astype(jnp.float32))[:, None]
    dc = (c - c0u.astype(jnp.float32))[:, None]
    Ia, Ib, Ic, Id = (x[:, None] for x in corners)
    out = Ia * (1 - dr) * (1 - dc) + Ib * (1 - dr) * dc + Ic * dr * (1 - dc) + Id * dr * dc
    return out[:, 0]


def _unpack_corners(vpack, hpack):
    ncorners = []
    pcorners = []
    for k in range(4):
        parts = []
        for x in range(3):
            idx = k * 3 + x
            parts.append(jnp.concatenate(
                [vpack[idx].reshape(-1), hpack[idx].reshape(-1)], axis=0))
        ncorners.append(jnp.stack(parts[:2], axis=1))
        pcorners.append(parts[2])
    return ncorners, pcorners


def _per_image_fields(pred, dv, grv, gcv, dh, grh, gch, vpack, hpack):
    h, w = pred.shape
    md = jnp.concatenate([dv.reshape(-1), dh.reshape(-1)], axis=0)
    gt_sel = jnp.stack(
        [jnp.concatenate([grv.reshape(-1), grh.reshape(-1)], axis=0),
         jnp.concatenate([gcv.reshape(-1), gch.reshape(-1)], axis=0)],
        axis=1)
    pred_zc, pmask = _extract_zero_crossings(pred)
    nv = (h - 1) * w
    r0f = jnp.floor(pred_zc[:, 0]).astype(jnp.int32)
    c0f = jnp.floor(pred_zc[:, 1]).astype(jnp.int32)
    eq_v = r0f[:nv].reshape(h - 1, w) == jnp.arange(h - 1, dtype=jnp.int32)[:, None] + 1
    eq_h = c0f[nv:].reshape(h, w - 1) == jnp.arange(w - 1, dtype=jnp.int32)[None, :] + 1
    ncorners, pcorners = _unpack_corners(vpack, hpack)
    sampled = _sample_normals(ncorners, pred_zc, h, w)
    keep = pmask & (md <= _DIST_THRESHOLD)
    dirv = gt_sel - pred_zc
    dot = jnp.sum(dirv * sampled, axis=1) * _UPDATE_SCALE
    r = pred_zc[:, 0]
    c = pred_zc[:, 1]
    r0 = jnp.floor(r).astype(jnp.int32)
    c0 = jnp.floor(c).astype(jnp.int32)
    r1 = r0 + 1
    c1 = c0 + 1
    ar = r - r0.astype(jnp.float32)
    ac = c - c0.astype(jnp.float32)
    corners = [(r0, c0, (1 - ar) * (1 - ac)), (r0, c1, (1 - ar) * ac),
               (r1, c0, ar * (1 - ac)), (r1, c1, ar * ac)]
    vals = []
    for rr, cc, wgt in corners:
        inb = (rr >= 0) & (rr < h) & (cc >= 0) & (cc < w)
        vals.append(jnp.where(keep & inb, dot * wgt, 0.0))
    lt_v = ~eq_v
    lt_h = ~eq_h
    v1g = vals[0][:nv].reshape(h - 1, w)
    v3g = vals[2][:nv].reshape(h - 1, w)
    h1g = vals[0][nv:].reshape(h, w - 1)
    h2g = vals[1][nv:].reshape(h, w - 1)
    T1 = jnp.pad(jnp.where(eq_v, v1g, 0.0), ((1, 0), (0, 0)))
    T2 = jnp.pad(jnp.where(lt_v, v1g, 0.0), ((0, 1), (0, 0)))
    T3 = jnp.pad(jnp.where(eq_h, h1g, 0.0), ((0, 0), (1, 0)))
    T4 = jnp.pad(jnp.where(lt_h, h1g, 0.0), ((0, 0), (0, 1)))
    T5 = jnp.pad(jnp.where(lt_h, h2g, 0.0), ((0, 0), (1, 0)))
    T6 = jnp.pad(jnp.where(lt_v, v3g, 0.0), ((1, 0), (0, 0)))
    dSDF = T1 + T2 + T3 + T4 + T5 + T6
    vals = _sample_pred_at_positions(pcorners, pred_zc)
    masked_vals = jnp.where(pmask, vals, 0.0)
    return dSDF, masked_vals


def kernel(pred_sdf, gt_sdf):
    outs = _nn_search(pred_sdf, gt_sdf)
    dSDFs, masked_vals = jax.vmap(_per_image_fields)(pred_sdf, *outs)
    injs = []
    pixs = []
    for b in range(pred_sdf.shape[0]):
        injs.append(jnp.sum(pred_sdf[b] * dSDFs[b]))
        pixs.append(jnp.sum(masked_vals[b]))
    inject = jnp.stack(injs).mean()
    pixel = jnp.stack(pixs).mean()
    return _W_INJECT * inject + _W_PIXEL * pixel


# speedup vs baseline: 2.0937x; 1.1484x over previous
import functools

import jax
import jax.numpy as jnp
from jax.experimental import pallas as pl

_EPS = 1e-08
_UPDATE_SCALE = 1.0
_DIST_THRESHOLD = 3.0
_W_INJECT = 1.0
_W_PIXEL = 1.0
_B = 4
_H = 64
_W = 64
_BIG = 1e20
_INVALID = 1e12


def _zc_grids(sdf):
    v1 = sdf[..., :-1, :]
    v2 = sdf[..., 1:, :]
    ii = jax.lax.broadcasted_iota(jnp.int32, v1.shape, v1.ndim - 2).astype(jnp.float32)
    c1 = v1 == 0.0
    c2 = jnp.logical_and(~c1, v2 == 0.0)
    c3 = jnp.logical_and(jnp.logical_and(~c1, ~c2), v1 * v2 < 0.0)
    alpha = jnp.abs(v1) / (jnp.abs(v1) + jnp.abs(v2) + _EPS)
    r_v = jnp.where(c1, ii, jnp.where(c2, ii + 1.0, ii + alpha))
    m_v = c1 | c2 | c3
    h1 = sdf[..., :, :-1]
    h2 = sdf[..., :, 1:]
    jj = jax.lax.broadcasted_iota(jnp.int32, h1.shape, h1.ndim - 1).astype(jnp.float32)
    d1 = h1 == 0.0
    d2 = jnp.logical_and(~d1, h2 == 0.0)
    d3 = jnp.logical_and(jnp.logical_and(~d1, ~d2), h1 * h2 < 0.0)
    beta = jnp.abs(h1) / (jnp.abs(h1) + jnp.abs(h2) + _EPS)
    c_h = jnp.where(d1, jj, jnp.where(d2, jj + 1.0, jj + beta))
    m_h = d1 | d2 | d3
    return (r_v, m_v), (c_h, m_h)


def _pad2(a, top, bottom, left, right, val):
    cfg = [(0, 0, 0)] * (a.ndim - 2) + [(top, bottom, 0), (left, right, 0)]
    return jax.lax.pad(a, jnp.float32(val), cfg)


def _nn_kernel(pred_ref, gt_ref, dv_ref, grv_ref, gcv_ref,
               dh_ref, grh_ref, gch_ref, vpack_ref, hpack_ref):
    pred = pred_ref[...]
    gt = gt_ref[...]
    (gv_r, gv_m), (gh_c, gh_m) = _zc_grids(gt)
    (pv_r, _), (ph_c, _) = _zc_grids(pred)

    GV = _pad2(jnp.where(gv_m, gv_r, _INVALID), 4, 4, 3, 4, _INVALID)
    GH = _pad2(jnp.where(gh_m, gh_c, _INVALID), 3, 4, 4, 4, _INVALID)

    iotaI_v = jax.lax.broadcasted_iota(jnp.int32, (_B, _H - 1, _W), 1).astype(jnp.float32)
    iotaJ_v = jax.lax.broadcasted_iota(jnp.int32, (_B, _H - 1, _W), 2).astype(jnp.float32)
    iotaI_h = jax.lax.broadcasted_iota(jnp.int32, (_B, _H, _W - 1), 1).astype(jnp.float32)
    iotaJ_h = jax.lax.broadcasted_iota(jnp.int32, (_B, _H, _W - 1), 2).astype(jnp.float32)

    def search(pr_r, pr_c, hh, ww, iI, iJ, vv, vh, v_aligned):
        best_d = jnp.full((_B, hh, ww), _BIG, jnp.float32)
        best_gr = jnp.zeros((_B, hh, ww), jnp.float32)
        best_gc = jnp.zeros((_B, hh, ww), jnp.float32)

        def upd(d, g_r, g_c, bd, bgr, bgc):
            better = d < bd
            return (jnp.where(better, d, bd),
                    jnp.where(better, g_r, bgr),
                    jnp.where(better, g_c, bgc))

        if v_aligned:
            bnd_vv = lambda di, dj: (max(0, abs(di) - 1), abs(dj))
            bnd_vh = lambda di, dj: (max(0, di - 1, -di), max(0, dj, -dj - 1))
        else:
            bnd_vv = lambda di, dj: (max(0, di, -di - 1), max(0, dj - 1, -dj))
            bnd_vh = lambda di, dj: (abs(di), max(0, abs(dj) - 1))

        dlo, dhi, jlo, jhi = vv
        g_cs = {dj: iJ + jnp.float32(dj) for dj in range(jlo, jhi + 1)}
        for di in range(dlo, dhi + 1):
            for dj in range(jlo, jhi + 1):
                a, b = bnd_vv(di, dj)
                if a * a + b * b > 9:
                    continue
                g_r = GV[:, 4 + di:4 + di + hh, 3 + dj:3 + dj + ww]
                g_c = g_cs[dj]
                dr = pr_r - g_r
                if v_aligned:
                    d = jnp.sqrt(dr * dr + jnp.float32(float(dj) * float(dj)))
                else:
                    dc = pr_c - g_c
                    d = jnp.sqrt(dr * dr + dc * dc)
                best_d, best_gr, best_gc = upd(d, g_r, g_c,
                                               best_d, best_gr, best_gc)
        dlo, dhi, jlo, jhi = vh
        g_rs = {di: iI + jnp.float32(di) for di in range(dlo, dhi + 1)}
        for di in range(dlo, dhi + 1):
            for dj in range(jlo, jhi + 1):
                a, b = bnd_vh(di, dj)
                if a * a + b * b > 9:
                    continue
                g_c = GH[:, 3 + di:3 + di + hh, 4 + dj:4 + dj + ww]
                g_r = g_rs[di]
                dc = pr_c - g_c
                if not v_aligned:
                    d = jnp.sqrt(jnp.float32(float(di) * float(di)) + dc * dc)
                else:
                    dr = pr_r - g_r
                    d = jnp.sqrt(dr * dr + dc * dc)
                best_d, best_gr, best_gc = upd(d, g_r, g_c,
                                               best_d, best_gr, best_gc)
        return best_d, best_gr, best_gc

    bd, bgr, bgc = search(pv_r, iotaJ_v, _H - 1, _W, iotaI_v, iotaJ_v,
                          (-4, 4, -3, 3), (-3, 4, -4, 3), True)
    dv_ref[...] = bd
    grv_ref[...] = bgr
    gcv_ref[...] = bgc

    bd, bgr, bgc = search(iotaI_h, ph_c, _H, _W - 1, iotaI_h, iotaJ_h,
                          (-4, 3, -3, 4), (-3, 3, -4, 4), False)
    dh_ref[...] = bd
    grh_ref[...] = bgr
    gch_ref[...] = bgc

    grad_r = jnp.concatenate(
        [pred[:, 1:2, :] - pred[:, 0:1, :],
         (pred[:, 2:, :] - pred[:, :-2, :]) / 2.0,
         pred[:, -1:, :] - pred[:, -2:-1, :]], axis=1)
    grad_c = jnp.concatenate(
        [pred[:, :, 1:2] - pred[:, :, 0:1],
         (pred[:, :, 2:] - pred[:, :, :-2]) / 2.0,
         pred[:, :, -1:] - pred[:, :, -2:-1]], axis=2)

    eq_v = pv_r == iotaI_v + 1.0
    eq_h = ph_c == iotaJ_h + 1.0

    def corner_v(X):
        Xc = jnp.concatenate([X[:, :, 1:], X[:, :, -1:]], axis=2)
        R2 = jnp.concatenate([X[:, 2:, :], X[:, -1:, :]], axis=1)
        C2 = jnp.concatenate([Xc[:, 2:, :], Xc[:, -1:, :]], axis=1)
        return (jnp.where(eq_v, X[:, 1:, :], X[:, :-1, :]),
                jnp.where(eq_v, Xc[:, 1:, :], Xc[:, :-1, :]),
                jnp.where(eq_v, R2, X[:, 1:, :]),
                jnp.where(eq_v, C2, Xc[:, 1:, :]))

    def corner_h(X):
        Xr = jnp.concatenate([X[:, 1:, :], X[:, -1:, :]], axis=1)
        A2 = jnp.concatenate([X[:, :, 2:], X[:, :, -1:]], axis=2)
        B2 = jnp.concatenate([Xr[:, :, 2:], Xr[:, :, -1:]], axis=2)
        return (jnp.where(eq_h, X[:, :, 1:], X[:, :, :-1]),
                jnp.where(eq_h, A2, X[:, :, 1:]),
                jnp.where(eq_h, Xr[:, :, 1:], Xr[:, :, :-1]),
                jnp.where(eq_h, B2, Xr[:, :, 1:]))

    vparts = []
    hparts = []
    for X in (grad_r, grad_c, pred):
        vparts.append(corner_v(X))
        hparts.append(corner_h(X))
    vpack_ref[...] = jnp.stack(
        [vparts[x][k] for k in range(4) for x in range(3)], axis=1)
    hpack_ref[...] = jnp.stack(
        [hparts[x][k] for k in range(4) for x in range(3)], axis=1)


@functools.partial(jax.jit, static_argnames=())
def _nn_search(pred_sdf, gt_sdf):
    shp_v = jax.ShapeDtypeStruct((_B, _H - 1, _W), jnp.float32)
    shp_h = jax.ShapeDtypeStruct((_B, _H, _W - 1), jnp.float32)
    pk_v = jax.ShapeDtypeStruct((_B, 12, _H - 1, _W), jnp.float32)
    pk_h = jax.ShapeDtypeStruct((_B, 12, _H, _W - 1), jnp.float32)
    return pl.pallas_call(
        _nn_kernel,
        out_shape=(shp_v, shp_v, shp_v, shp_h, shp_h, shp_h, pk_v, pk_h),
    )(pred_sdf, gt_sdf)


def _extract_zero_crossings(sdf):
    h, w = sdf.shape
    (r_v, m_v), (c_h, m_h) = _zc_grids(sdf)
    ii2 = jnp.broadcast_to(jnp.arange(h, dtype=jnp.float32)[:, None], (h, w - 1))
    jj = jnp.broadcast_to(jnp.arange(w, dtype=jnp.float32)[None, :], (h - 1, w))
    pts_v = jnp.stack([r_v.reshape(-1), jj.reshape(-1)], axis=1)
    pts_h = jnp.stack([ii2.reshape(-1), c_h.reshape(-1)], axis=1)
    return (jnp.concatenate([pts_v, pts_h], axis=0),
            jnp.concatenate([m_v.reshape(-1), m_h.reshape(-1)], axis=0))


def _compute_normals(sdf):
    grad_r = jnp.zeros_like(sdf)
    grad_r = grad_r.at[1:-1].set((sdf[2:] - sdf[:-2]) / 2.0)
    grad_r = grad_r.at[0].set(sdf[1] - sdf[0])
    grad_r = grad_r.at[-1].set(sdf[-1] - sdf[-2])
    grad_c = jnp.zeros_like(sdf)
    grad_c = grad_c.at[:, 1:-1].set((sdf[:, 2:] - sdf[:, :-2]) / 2.0)
    grad_c = grad_c.at[:, 0].set(sdf[:, 1] - sdf[:, 0])
    grad_c = grad_c.at[:, -1].set(sdf[:, -1] - sdf[:, -2])
    return jnp.stack([grad_r, grad_c], axis=2)


def _sample_normals(corners, pts, h, w):
    r = pts[:, 0]
    c = pts[:, 1]
    r0u = jnp.floor(r).astype(jnp.int32)
    c0u = jnp.floor(c).astype(jnp.int32)
    r0 = jnp.clip(r0u, 0, h - 1)
    c0 = jnp.clip(c0u, 0, w - 1)
    ar = (r - r0.astype(jnp.float32))[:, None]
    ac = (c - c0.astype(jnp.float32))[:, None]
    Ia, Ib, Ic, Id = corners
    n = Ia * (1 - ar) * (1 - ac) + Ib * (1 - ar) * ac + Ic * ar * (1 - ac) + Id * ar * ac
    return n / (jnp.linalg.norm(n, axis=1, keepdims=True) + 1e-08)


def _sample_pred_at_positions(corners, positions):
    r = positions[:, 0]
    c = positions[:, 1]
    r0u = jnp.floor(r).astype(jnp.int32)
    c0u = jnp.floor(c).astype(jnp.int32)
    dr = (r - r0u.astype(jnp.float32))[:, None]
    dc = (c - c0u.astype(jnp.float32))[:, None]
    Ia, Ib, Ic, Id = (x[:, None] for x in corners)
    out = Ia * (1 - dr) * (1 - dc) + Ib * (1 - dr) * dc + Ic * dr * (1 - dc) + Id * dr * dc
    return out[:, 0]


def _unpack_corners(vpack, hpack):
    ncorners = []
    pcorners = []
    for k in range(4):
        parts = []
        for x in range(3):
            idx = k * 3 + x
            parts.append(jnp.concatenate(
                [vpack[idx].reshape(-1), hpack[idx].reshape(-1)], axis=0))
        ncorners.append(jnp.stack(parts[:2], axis=1))
        pcorners.append(parts[2])
    return ncorners, pcorners


def _per_image_fields(pred, dv, grv, gcv, dh, grh, gch, vpack, hpack):
    h, w = pred.shape
    md = jnp.concatenate([dv.reshape(-1), dh.reshape(-1)], axis=0)
    gt_sel = jnp.stack(
        [jnp.concatenate([grv.reshape(-1), grh.reshape(-1)], axis=0),
         jnp.concatenate([gcv.reshape(-1), gch.reshape(-1)], axis=0)],
        axis=1)
    pred_zc, pmask = _extract_zero_crossings(pred)
    nv = (h - 1) * w
    r0f = jnp.floor(pred_zc[:, 0]).astype(jnp.int32)
    c0f = jnp.floor(pred_zc[:, 1]).astype(jnp.int32)
    eq_v = r0f[:nv].reshape(h - 1, w) == jnp.arange(h - 1, dtype=jnp.int32)[:, None] + 1
    eq_h = c0f[nv:].reshape(h, w - 1) == jnp.arange(w - 1, dtype=jnp.int32)[None, :] + 1
    ncorners, pcorners = _unpack_corners(vpack, hpack)
    sampled = _sample_normals(ncorners, pred_zc, h, w)
    keep = pmask & (md <= _DIST_THRESHOLD)
    dirv = gt_sel - pred_zc
    dot = jnp.sum(dirv * sampled, axis=1) * _UPDATE_SCALE
    r = pred_zc[:, 0]
    c = pred_zc[:, 1]
    r0 = jnp.floor(r).astype(jnp.int32)
    c0 = jnp.floor(c).astype(jnp.int32)
    r1 = r0 + 1
    c1 = c0 + 1
    ar = r - r0.astype(jnp.float32)
    ac = c - c0.astype(jnp.float32)
    corners = [(r0, c0, (1 - ar) * (1 - ac)), (r0, c1, (1 - ar) * ac),
               (r1, c0, ar * (1 - ac)), (r1, c1, ar * ac)]
    vals = []
    for rr, cc, wgt in corners:
        inb = (rr >= 0) & (rr < h) & (cc >= 0) & (cc < w)
        vals.append(jnp.where(keep & inb, dot * wgt, 0.0))
    lt_v = ~eq_v
    lt_h = ~eq_h
    v1g = vals[0][:nv].reshape(h - 1, w)
    v3g = vals[2][:nv].reshape(h - 1, w)
    h1g = vals[0][nv:].reshape(h, w - 1)
    h2g = vals[1][nv:].reshape(h, w - 1)
    T1 = jnp.pad(jnp.where(eq_v, v1g, 0.0), ((1, 0), (0, 0)))
    T2 = jnp.pad(jnp.where(lt_v, v1g, 0.0), ((0, 1), (0, 0)))
    T3 = jnp.pad(jnp.where(eq_h, h1g, 0.0), ((0, 0), (1, 0)))
    T4 = jnp.pad(jnp.where(lt_h, h1g, 0.0), ((0, 0), (0, 1)))
    T5 = jnp.pad(jnp.where(lt_h, h2g, 0.0), ((0, 0), (1, 0)))
    T6 = jnp.pad(jnp.where(lt_v, v3g, 0.0), ((1, 0), (0, 0)))
    dSDF = T1 + T2 + T3 + T4 + T5 + T6
    vals = _sample_pred_at_positions(pcorners, pred_zc)
    masked_vals = jnp.where(pmask, vals, 0.0)
    return dSDF, masked_vals


def kernel(pred_sdf, gt_sdf):
    outs = _nn_search(pred_sdf, gt_sdf)
    dSDFs, masked_vals = jax.vmap(_per_image_fields)(pred_sdf, *outs)
    injs = []
    pixs = []
    for b in range(pred_sdf.shape[0]):
        injs.append(jnp.sum(pred_sdf[b] * dSDFs[b]))
        pixs.append(jnp.sum(masked_vals[b]))
    inject = jnp.stack(injs).mean()
    pixel = jnp.stack(pixs).mean()
    return _W_INJECT * inject + _W_PIXEL * pixel
